# baseline (speedup 1.0000x reference)
import functools
import jax
import jax.numpy as jnp
from jax import lax
from jax.experimental import pallas as pl
from jax.experimental.pallas import tpu as pltpu
from jax.experimental.pallas import tpu_sc as plsc

_PADDING_IDX = 0
_NC = 2
_NS = 16
_NW = _NC * _NS
_L = 16
_C = 32
_NBUF = 3


_GATHER_DNUMS = lax.GatherDimensionNumbers(
    offset_dims=(), collapsed_slice_dims=(0,), start_index_map=(0,))


def _rot_take(m, rot):
    return lax.gather(
        m, rot[:, None], dimension_numbers=_GATHER_DNUMS, slice_sizes=(1,),
        mode=lax.GatherScatterMode.PROMISE_IN_BOUNDS)


def _any_pad(v):
    lane = lax.iota(jnp.int32, _L)
    m = jnp.where(v == _PADDING_IDX, 1, 0)
    for k in (8, 4, 2, 1):
        rot = (lane + k) & (_L - 1)
        m = m | _rot_take(m, rot)
    return m[0]


def _sc_body(seq_len, dim, in_hbm, w_hbm, out_hbm,
             ids_v, w_v0, w_v1, w_v2, zero_v,
             sem_i, sem_r0, sem_r1, sem_r2, sem_w0, sem_w1, sem_w2):
    bsz = 4
    rows_per_w = seq_len // _NW
    n_chunks = rows_per_w // _C
    wid = lax.axis_index("s") * _NC + lax.axis_index("c")
    base0 = wid * rows_per_w

    i_handles = [
        pltpu.async_copy(in_hbm.at[pl.ds(b * seq_len + base0, rows_per_w)],
                         ids_v.at[pl.ds(b * rows_per_w, rows_per_w)], sem_i)
        for b in range(bsz)]

    for j in range(dim // _L):
        zero_v[0, pl.ds(j * _L, _L)] = jnp.zeros((_L,), jnp.float32)

    bufs = (w_v0, w_v1, w_v2)
    sems_r = (sem_r0, sem_r1, sem_r2)
    sems_w = (sem_w0, sem_w1, sem_w2)

    def fixup(c):
        for b in range(bsz):
            for g in range(_C // _L):
                row0 = b * rows_per_w + c * _C + g * _L
                v = ids_v[pl.ds(row0, _L)]

                @pl.when(_any_pad(v) == 1)
                def _():
                    def row_body(r, _):
                        vr = ids_v[pl.ds(row0 + r, _L)]

                        @pl.when(vr[0] == _PADDING_IDX)
                        def _():
                            pltpu.sync_copy(
                                zero_v,
                                out_hbm.at[pl.ds(
                                    b * seq_len + base0 + c * _C + g * _L + r,
                                    1)])
                        return 0

                    lax.fori_loop(0, _L, row_body, 0)

    def read(c):
        return pltpu.async_copy(
            w_hbm.at[pl.ds(base0 + c * _C, _C)],
            bufs[c % _NBUF], sems_r[c % _NBUF])

    r_handles = {0: read(0)}
    if n_chunks > 1:
        r_handles[1] = read(1)
    w_handles = {}
    ids_ready = False
    for c in range(n_chunks):
        r_handles[c].wait()
        w_handles[c] = [
            pltpu.async_copy(
                bufs[c % _NBUF],
                out_hbm.at[pl.ds(b * seq_len + base0 + c * _C, _C)],
                sems_w[c % _NBUF])
            for b in range(bsz)]
        if c == 0:
            if 2 < n_chunks:
                r_handles[2] = read(2)
        if c >= 1:
            for h in w_handles[c - 1]:
                h.wait()
            if c + 2 < n_chunks and c >= 1:
                r_handles[c + 2] = read(c + 2)
            if not ids_ready:
                for h in i_handles:
                    h.wait()
                ids_ready = True
            fixup(c - 1)
    for h in w_handles[n_chunks - 1]:
        h.wait()
    fixup(n_chunks - 1)


def kernel(input, weights):
    bsz, seq_len = input.shape
    dim = weights.shape[1]
    mesh = plsc.VectorSubcoreMesh(
        core_axis_name="c", subcore_axis_name="s",
        num_cores=_NC, num_subcores=_NS)
    inp_flat = input.reshape(bsz * seq_len)
    body = functools.partial(_sc_body, seq_len, dim)
    out = pl.kernel(
        body,
        out_type=jax.ShapeDtypeStruct((bsz * seq_len, dim), weights.dtype),
        mesh=mesh,
        scratch_types=[
            pltpu.VMEM((bsz * (seq_len // _NW) + _L,), jnp.int32),
            pltpu.VMEM((_C, dim), jnp.float32),
            pltpu.VMEM((_C, dim), jnp.float32),
            pltpu.VMEM((_C, dim), jnp.float32),
            pltpu.VMEM((1, dim), jnp.float32),
            pltpu.SemaphoreType.DMA,
            pltpu.SemaphoreType.DMA,
            pltpu.SemaphoreType.DMA,
            pltpu.SemaphoreType.DMA,
            pltpu.SemaphoreType.DMA,
            pltpu.SemaphoreType.DMA,
            pltpu.SemaphoreType.DMA,
        ],
    )(inp_flat, weights)
    return out.reshape(bsz, seq_len, dim)


# speedup vs baseline: 1.4128x; 1.4128x over previous
import jax
import jax.numpy as jnp
from jax.experimental import pallas as pl

_PADDING_IDX = 0
_S_BLK = 512


def _body(in_ref, w_ref, out_ref):
    w = w_ref[...]
    for b in range(out_ref.shape[0]):
        mask = in_ref[:, b:b + 1] != _PADDING_IDX
        out_ref[b] = jnp.where(mask, w, 0.0)


def kernel(input, weights):
    bsz, seq_len = input.shape
    dim = weights.shape[1]
    num_s = seq_len // _S_BLK
    inp_t = input.T
    return pl.pallas_call(
        _body,
        grid=(num_s,),
        in_specs=[
            pl.BlockSpec((_S_BLK, bsz), lambda s: (s, 0)),
            pl.BlockSpec((_S_BLK, dim), lambda s: (s, 0)),
        ],
        out_specs=pl.BlockSpec((bsz, _S_BLK, dim), lambda s: (0, s, 0)),
        out_shape=jax.ShapeDtypeStruct((bsz, seq_len, dim), weights.dtype),
    )(inp_t, weights)


# speedup vs baseline: 1.4487x; 1.0254x over previous
import jax
import jax.numpy as jnp
from jax.experimental import pallas as pl

_PADDING_IDX = 0
_S_BLK = 1024


def _body(in_ref, w_ref, out_ref):
    w = w_ref[...]
    for b in range(out_ref.shape[0]):
        mask = in_ref[:, b:b + 1] != _PADDING_IDX
        out_ref[b] = jnp.where(mask, w, 0.0)


def kernel(input, weights):
    bsz, seq_len = input.shape
    dim = weights.shape[1]
    num_s = seq_len // _S_BLK
    inp_t = input.T
    return pl.pallas_call(
        _body,
        grid=(num_s,),
        in_specs=[
            pl.BlockSpec((_S_BLK, bsz), lambda s: (s, 0)),
            pl.BlockSpec((_S_BLK, dim), lambda s: (s, 0)),
        ],
        out_specs=pl.BlockSpec((bsz, _S_BLK, dim), lambda s: (0, s, 0)),
        out_shape=jax.ShapeDtypeStruct((bsz, seq_len, dim), weights.dtype),
    )(inp_t, weights)
